# Initial kernel scaffold; baseline (speedup 1.0000x reference)
import jax
import jax.numpy as jnp
from jax.experimental import pallas as pl


def kernel(nodes, adj, features):
    raise NotImplementedError("write your pallas kernel here")


# speedup vs baseline: 1.2191x; 1.2191x over previous
import functools

import jax
import jax.numpy as jnp
from jax import lax
from jax.experimental import pallas as pl
from jax.experimental.pallas import tpu as pltpu
from jax.experimental.pallas import tpu_sc as plsc

B = 10000
DEG = 32
N = 50000
D = 128
K = DEG + 1
L = 16
NC, NS = 2, 16
NW = NC * NS
G = 64
BPAD = ((B + NW * G - 1) // (NW * G)) * (NW * G)
BPW = BPAD // NW
NG = BPW // G
NR = NG * K
VSEG = D // L


def _sc_aggregate(idx, features):
    mesh = plsc.VectorSubcoreMesh(
        core_axis_name="c", subcore_axis_name="s", num_cores=NC, num_subcores=NS
    )

    @functools.partial(
        pl.kernel,
        out_type=jax.ShapeDtypeStruct((BPAD, D), jnp.float32),
        mesh=mesh,
        scratch_types=[
            pltpu.VMEM((NR, G), jnp.int32),
            pltpu.VMEM((G, D), jnp.float32),
            pltpu.VMEM((G, D), jnp.float32),
            pltpu.SemaphoreType.DMA,
        ],
    )
    def body(idx_hbm, feat_hbm, out_hbm, idx_v, buf_v, acc_v, sem):
        w = lax.axis_index("s") * NC + lax.axis_index("c")
        pltpu.sync_copy(idx_hbm.at[w], idx_v)
        scale = jnp.float32(1.0 / K)

        def group_body(g, carry):
            r0 = g * K
            pltpu.async_copy(feat_hbm.at[idx_v.at[r0]], acc_v, sem).wait()

            def slot_body(j, carry):
                pltpu.async_copy(feat_hbm.at[idx_v.at[r0 + j]], buf_v, sem).wait()

                def row_body(c, carry):
                    for v in range(VSEG):
                        sl = pl.ds(v * L, L)
                        plsc.addupdate(acc_v.at[c, sl], buf_v[c, sl])
                    return carry

                return lax.fori_loop(0, G, row_body, carry)

            lax.fori_loop(1, K, slot_body, 0)

            def scale_body(c, carry):
                for v in range(VSEG):
                    sl = pl.ds(v * L, L)
                    acc_v[c, sl] = acc_v[c, sl] * scale
                return carry

            lax.fori_loop(0, G, scale_body, 0)
            pltpu.sync_copy(acc_v, out_hbm.at[pl.ds(w * BPW + g * G, G)])
            return carry

        lax.fori_loop(0, NG, group_body, 0)

    return body(idx, features)


def kernel(nodes, adj, features):
    nodes = nodes.astype(jnp.int32)
    adj = adj.astype(jnp.int32)
    idx = jnp.concatenate([nodes[:, None], adj], axis=1)
    idx = jnp.pad(idx, ((0, BPAD - B), (0, 0)))
    idx = idx.reshape(NW, NG, G, K).transpose(0, 1, 3, 2).reshape(NW, NR, G)
    out = _sc_aggregate(idx, features)
    return out[:B]


# speedup vs baseline: 1.5734x; 1.2907x over previous
import functools

import jax
import jax.numpy as jnp
from jax import lax
from jax.experimental import pallas as pl
from jax.experimental.pallas import tpu as pltpu
from jax.experimental.pallas import tpu_sc as plsc

B = 10000
DEG = 32
N = 50000
D = 128
K = DEG + 1
L = 16
NC, NS = 2, 16
NW = NC * NS
G = 64
BPAD = ((B + NW * G - 1) // (NW * G)) * (NW * G)
BPW = BPAD // NW
NG = BPW // G
NR = NG * K
VSEG = D // L
NBUF = 3
CU = 4


def _sc_aggregate(idx, features):
    mesh = plsc.VectorSubcoreMesh(
        core_axis_name="c", subcore_axis_name="s", num_cores=NC, num_subcores=NS
    )

    @functools.partial(
        pl.kernel,
        out_type=jax.ShapeDtypeStruct((BPAD, D), jnp.float32),
        mesh=mesh,
        scratch_types=[
            pltpu.VMEM((NR, G), jnp.int32),
            [pltpu.VMEM((G, D), jnp.float32) for _ in range(NBUF)],
            pltpu.VMEM((G, D), jnp.float32),
            [pltpu.SemaphoreType.DMA for _ in range(NBUF)],
        ],
    )
    def body(idx_hbm, feat_hbm, out_hbm, idx_v, bufs, acc_v, sems):
        w = lax.axis_index("s") * NC + lax.axis_index("c")
        pltpu.sync_copy(idx_hbm.at[w], idx_v)
        scale = jnp.float32(1.0 / K)
        zeros = jnp.zeros((L,), jnp.float32)

        for g in range(NG):
            r0 = g * K

            def zero_body(c4, carry):
                for cc in range(CU):
                    for v in range(VSEG):
                        acc_v[c4 * CU + cc, pl.ds(v * L, L)] = zeros
                return carry

            lax.fori_loop(0, G // CU, zero_body, 0)

            for k in range(NBUF):
                pltpu.async_copy(feat_hbm.at[idx_v.at[r0 + k]], bufs[k], sems[k])

            def t_body(t, carry):
                for k in range(NBUF):
                    pltpu.make_async_copy(
                        feat_hbm.at[idx_v.at[r0]], bufs[k], sems[k]
                    ).wait()

                    def acc_body(c4, carry2, k=k):
                        for cc in range(CU):
                            for v in range(VSEG):
                                sl = pl.ds(v * L, L)
                                plsc.addupdate(
                                    acc_v.at[c4 * CU + cc, sl],
                                    bufs[k][c4 * CU + cc, sl],
                                )
                        return carry2

                    lax.fori_loop(0, G // CU, acc_body, 0)
                    nxt = NBUF * t + k + NBUF

                    @pl.when(nxt < K)
                    def _(k=k, nxt=nxt):
                        pltpu.async_copy(
                            feat_hbm.at[idx_v.at[r0 + nxt]], bufs[k], sems[k]
                        )

                return carry

            lax.fori_loop(0, K // NBUF, t_body, 0)

            def scale_body(c4, carry):
                for cc in range(CU):
                    for v in range(VSEG):
                        sl = pl.ds(v * L, L)
                        acc_v[c4 * CU + cc, sl] = acc_v[c4 * CU + cc, sl] * scale
                return carry

            lax.fori_loop(0, G // CU, scale_body, 0)
            pltpu.sync_copy(acc_v, out_hbm.at[pl.ds(w * BPW + g * G, G)])

    return body(idx, features)


def kernel(nodes, adj, features):
    nodes = nodes.astype(jnp.int32)
    adj = adj.astype(jnp.int32)
    idx = jnp.concatenate([nodes[:, None], adj], axis=1)
    idx = jnp.pad(idx, ((0, BPAD - B), (0, 0)))
    idx = idx.reshape(NW, NG, G, K).transpose(0, 1, 3, 2).reshape(NW, NR, G)
    out = _sc_aggregate(idx, features)
    return out[:B]


# speedup vs baseline: 1.6033x; 1.0190x over previous
import functools

import jax
import jax.numpy as jnp
from jax import lax
from jax.experimental import pallas as pl
from jax.experimental.pallas import tpu as pltpu
from jax.experimental.pallas import tpu_sc as plsc

B = 10000
DEG = 32
N = 50000
D = 128
K = DEG + 1
L = 16
NC, NS = 2, 16
NW = NC * NS
G = 64
BPAD = ((B + NW * G - 1) // (NW * G)) * (NW * G)
BPW = BPAD // NW
NG = BPW // G
NR = NG * K
VSEG = D // L
NACC = 4
CU = 4


def _sc_aggregate(idx, features):
    mesh = plsc.VectorSubcoreMesh(
        core_axis_name="c", subcore_axis_name="s", num_cores=NC, num_subcores=NS
    )

    @functools.partial(
        pl.kernel,
        out_type=jax.ShapeDtypeStruct((BPAD, D), jnp.float32),
        mesh=mesh,
        scratch_types=[
            pltpu.VMEM((NR, G), jnp.int32),
            [pltpu.VMEM((G, D), jnp.float32) for _ in range(NACC)],
            [pltpu.SemaphoreType.DMA for _ in range(NACC)],
        ],
    )
    def body(idx_hbm, feat_hbm, out_hbm, idx_v, accs, sems):
        w = lax.axis_index("s") * NC + lax.axis_index("c")
        pltpu.sync_copy(idx_hbm.at[w], idx_v)
        scale = jnp.float32(1.0 / K)
        nfull = (K - NACC) // NACC
        ntail = (K - NACC) % NACC

        for g in range(NG):
            r0 = g * K
            for k in range(NACC):
                pltpu.async_copy(feat_hbm.at[idx_v.at[r0 + k]], accs[k], sems[k])

            def t_body(t, carry):
                for k in range(NACC):
                    pltpu.make_async_copy(
                        feat_hbm.at[idx_v.at[r0]], accs[k], sems[k]
                    ).wait()
                    j = NACC + NACC * t + k
                    pltpu.async_copy(
                        feat_hbm.at[idx_v.at[r0 + j]], accs[k], sems[k], add=True
                    )
                return carry

            lax.fori_loop(0, nfull, t_body, 0)
            for k in range(ntail):
                pltpu.make_async_copy(
                    feat_hbm.at[idx_v.at[r0]], accs[k], sems[k]
                ).wait()
                j = NACC + NACC * nfull + k
                pltpu.async_copy(
                    feat_hbm.at[idx_v.at[r0 + j]], accs[k], sems[k], add=True
                )
            for k in range(NACC):
                pltpu.make_async_copy(
                    feat_hbm.at[idx_v.at[r0]], accs[k], sems[k]
                ).wait()

            def red_body(c4, carry):
                for cc in range(CU):
                    for v in range(VSEG):
                        sl = pl.ds(v * L, L)
                        s = accs[0][c4 * CU + cc, sl]
                        for k in range(1, NACC):
                            s = s + accs[k][c4 * CU + cc, sl]
                        accs[0][c4 * CU + cc, sl] = s * scale
                return carry

            lax.fori_loop(0, G // CU, red_body, 0)
            pltpu.sync_copy(accs[0], out_hbm.at[pl.ds(w * BPW + g * G, G)])

    return body(idx, features)


def kernel(nodes, adj, features):
    nodes = nodes.astype(jnp.int32)
    adj = adj.astype(jnp.int32)
    idx = jnp.concatenate([nodes[:, None], adj], axis=1)
    idx = jnp.pad(idx, ((0, BPAD - B), (0, 0)))
    idx = idx.reshape(NW, NG, G, K).transpose(0, 1, 3, 2).reshape(NW, NR, G)
    out = _sc_aggregate(idx, features)
    return out[:B]
